# baseline (speedup 1.0000x reference)
import functools

import jax
import jax.numpy as jnp
from jax import lax
from jax.experimental import pallas as pl
from jax.experimental.pallas import tpu as pltpu
from jax.experimental.pallas import tpu_sc as plsc

B = 4
T = 2048
D = 1024
NC = 2
NS = 16
NW = NC * NS
T_PER_W = T // NW
CT = 16
NTC = T_PER_W // CT
NG = NTC * B
NBUF = 4
LANES = 16

_mesh = plsc.VectorSubcoreMesh(core_axis_name="c", subcore_axis_name="s")


@functools.partial(
    pl.kernel,
    mesh=_mesh,
    out_type=jax.ShapeDtypeStruct((B * T, D), jnp.float32),
    scratch_types=[
        pltpu.VMEM((NG, CT), jnp.int32),
        pltpu.VMEM((CT, D), jnp.float32),
        pltpu.VMEM((CT, D), jnp.float32),
    ]
    + [pltpu.VMEM((CT, D), jnp.float32) for _ in range(NBUF)]
    + [pltpu.SemaphoreType.DMA for _ in range(2 + 2 * NBUF)],
)
def _embed(idx_hbm, tok_hbm, pos_hbm, out_hbm, idx_v, *rest):
    posb = rest[0:2]
    toks = rest[2:2 + NBUF]
    psem = rest[2 + NBUF:4 + NBUF]
    gsem = rest[4 + NBUF:4 + 2 * NBUF]
    ssem = rest[4 + 2 * NBUF:]
    wid = lax.axis_index("s") * NC + lax.axis_index("c")
    t0 = wid * T_PER_W

    idx_handles = []
    for tc in range(NTC):
        for b in range(B):
            idx_handles.append(pltpu.async_copy(
                idx_hbm.at[pl.ds(b * T + t0 + tc * CT, CT)],
                idx_v.at[tc * B + b], gsem[0]))
    for h in idx_handles:
        h.wait()

    def issue_pos(tc):
        return pltpu.async_copy(
            pos_hbm.at[pl.ds(t0 + tc * CT, CT)], posb[tc % 2], psem[tc % 2])

    def issue_gather(g):
        k = g % NBUF
        return pltpu.async_copy(tok_hbm.at[idx_v.at[g]], toks[k], gsem[k])

    pos_handles = {0: issue_pos(0)}
    gathers = {g: issue_gather(g) for g in range(NBUF - 1)}
    stores = {}
    for g in range(NG):
        k = g % NBUF
        tc, b = divmod(g, B)
        if b == 0:
            pos_handles[tc].wait()
            if tc + 1 < NTC:
                pos_handles[tc + 1] = issue_pos(tc + 1)
        gathers[g].wait()
        tok = toks[k]
        pos = posb[tc % 2]

        def row_body(r, carry, tok=tok, pos=pos):
            for j in range(D // LANES):
                sl = pl.ds(j * LANES, LANES)
                tok[r, sl] += pos[r, sl]
            return carry

        lax.fori_loop(0, CT, row_body, 0)
        stores[g] = pltpu.async_copy(
            tok, out_hbm.at[pl.ds(b * T + t0 + tc * CT, CT)], ssem[k])
        nxt = g + NBUF - 1
        if nxt < NG:
            if g >= 1:
                stores[g - 1].wait()
            gathers[nxt] = issue_gather(nxt)
    for g in range(max(0, NG - NBUF), NG):
        stores[g].wait()


def kernel(idx, token_embedding_table, position_embedding_table):
    idx_flat = idx.astype(jnp.int32).reshape(B * T)
    out = _embed(idx_flat, token_embedding_table, position_embedding_table)
    return out.reshape(B, T, D)


# speedup vs baseline: 1.0222x; 1.0222x over previous
import functools

import jax
import jax.numpy as jnp
from jax import lax
from jax.experimental import pallas as pl
from jax.experimental.pallas import tpu as pltpu
from jax.experimental.pallas import tpu_sc as plsc

B = 4
T = 2048
D = 1024
NC = 2
NS = 16
NW = NC * NS
T_PER_W = T // NW
CT = 16
NTC = T_PER_W // CT
NG = NTC * B
NBUF = 4
LANES = 16

_mesh = plsc.VectorSubcoreMesh(core_axis_name="c", subcore_axis_name="s")


@functools.partial(
    pl.kernel,
    mesh=_mesh,
    out_type=jax.ShapeDtypeStruct((B * T, D), jnp.float32),
    scratch_types=[
        pltpu.VMEM((NG, CT), jnp.int32),
        pltpu.VMEM((CT, D), jnp.float32),
        pltpu.VMEM((CT, D), jnp.float32),
    ]
    + [pltpu.VMEM((CT, D), jnp.float32) for _ in range(NBUF)]
    + [pltpu.SemaphoreType.DMA for _ in range(2 + 2 * NBUF)],
)
def _embed(idx_hbm, tok_hbm, pos_hbm, out_hbm, idx_v, *rest):
    posb = rest[0:2]
    toks = rest[2:2 + NBUF]
    psem = rest[2 + NBUF:4 + NBUF]
    gsem = rest[4 + NBUF:4 + 2 * NBUF]
    ssem = rest[4 + 2 * NBUF:]
    wid = lax.axis_index("s") * NC + lax.axis_index("c")
    t0 = wid * T_PER_W

    idx_handles = []
    for tc in range(NTC):
        for b in range(B):
            idx_handles.append(pltpu.async_copy(
                idx_hbm.at[pl.ds(b * T + t0 + tc * CT, CT)],
                idx_v.at[tc * B + b], gsem[0]))
    for h in idx_handles:
        h.wait()

    def issue_pos(tc):
        return pltpu.async_copy(
            pos_hbm.at[pl.ds(t0 + tc * CT, CT)], posb[tc % 2], psem[tc % 2])

    def issue_gather(g):
        k = g % NBUF
        return pltpu.async_copy(tok_hbm.at[idx_v.at[g]], toks[k], gsem[k])

    pos_handles = {0: issue_pos(0)}
    gathers = {g: issue_gather(g) for g in range(NBUF - 1)}
    stores = {}
    for g in range(NG):
        k = g % NBUF
        tc, b = divmod(g, B)
        if b == 0:
            pos_handles[tc].wait()
            if tc + 1 < NTC:
                pos_handles[tc + 1] = issue_pos(tc + 1)
        gathers[g].wait()
        tok = toks[k]
        pos = posb[tc % 2]

        @plsc.parallel_loop(0, CT, 1)
        def _add_rows(r, tok=tok, pos=pos):
            for j in range(D // LANES):
                sl = pl.ds(j * LANES, LANES)
                tok[r, sl] += pos[r, sl]
        stores[g] = pltpu.async_copy(
            tok, out_hbm.at[pl.ds(b * T + t0 + tc * CT, CT)], ssem[k])
        nxt = g + NBUF - 1
        if nxt < NG:
            if g >= 1:
                stores[g - 1].wait()
            gathers[nxt] = issue_gather(nxt)
    for g in range(max(0, NG - NBUF), NG):
        stores[g].wait()


def kernel(idx, token_embedding_table, position_embedding_table):
    idx_flat = idx.astype(jnp.int32).reshape(B * T)
    out = _embed(idx_flat, token_embedding_table, position_embedding_table)
    return out.reshape(B, T, D)


# speedup vs baseline: 1.1853x; 1.1597x over previous
import functools

import jax
import jax.numpy as jnp
from jax import lax
from jax.experimental import pallas as pl
from jax.experimental.pallas import tpu as pltpu
from jax.experimental.pallas import tpu_sc as plsc

B = 4
T = 2048
D = 1024
NC = 2
NS = 16
NW = NC * NS
T_PER_W = T // NW
CT = 8
NTC = T_PER_W // CT
LANES = 16

_mesh = plsc.VectorSubcoreMesh(core_axis_name="c", subcore_axis_name="s")


@functools.partial(
    pl.kernel,
    mesh=_mesh,
    out_type=jax.ShapeDtypeStruct((B * T, D), jnp.float32),
    scratch_types=[
        pltpu.VMEM((B, T_PER_W), jnp.int32),
        pltpu.VMEM((CT, D), jnp.float32),
        pltpu.VMEM((CT, D), jnp.float32),
    ]
    + [pltpu.VMEM((CT, D), jnp.float32) for _ in range(2 * B)]
    + [pltpu.SemaphoreType.DMA for _ in range(2 + 2 * 2 * B + 1)],
)
def _embed(idx_hbm, tok_hbm, pos_hbm, out_hbm, idx_v, *rest):
    posb = rest[0:2]
    toks = rest[2:2 + 2 * B]
    psem = rest[2 + 2 * B:4 + 2 * B]
    gsem = rest[4 + 2 * B:4 + 4 * B]
    ssem = rest[4 + 4 * B:4 + 6 * B]
    isem = rest[4 + 6 * B]
    wid = lax.axis_index("s") * NC + lax.axis_index("c")
    t0 = wid * T_PER_W

    idx_handles = [
        pltpu.async_copy(idx_hbm.at[pl.ds(b * T + t0, T_PER_W)],
                         idx_v.at[b], isem)
        for b in range(B)
    ]
    for h in idx_handles:
        h.wait()

    def issue_pos(p):
        return pltpu.async_copy(
            pos_hbm.at[pl.ds(t0 + p * CT, CT)], posb[p % 2], psem[p % 2])

    def issue_gather(p, b):
        k = (p % 2) * B + b
        return pltpu.async_copy(
            tok_hbm.at[idx_v.at[b].at[pl.ds(p * CT, CT)]], toks[k], gsem[k])

    pos_handles = {0: issue_pos(0)}
    gathers = {(0, b): issue_gather(0, b) for b in range(B)}
    stores = {}
    for p in range(NTC):
        s = (p % 2) * B
        pos_handles[p].wait()
        if p + 1 < NTC:
            pos_handles[p + 1] = issue_pos(p + 1)
        for b in range(B):
            gathers[(p, b)].wait()
        if p + 1 < NTC:
            for b in range(B):
                if p >= 1:
                    stores[(p - 1, b)].wait()
                gathers[(p + 1, b)] = issue_gather(p + 1, b)
        t_s, pos = toks[s:s + B], posb[p % 2]

        @plsc.parallel_loop(0, CT, 1)
        def _add_rows(r, t_s=t_s, pos=pos):
            @plsc.parallel_loop(0, D, LANES, unroll=4)
            def _add_lanes(c, r=r, t_s=t_s, pos=pos):
                sl = pl.ds(c, LANES)
                pv = pos[r, sl]
                for b in range(B):
                    t_s[b][r, sl] += pv

        for b in range(B):
            stores[(p, b)] = pltpu.async_copy(
                toks[s + b], out_hbm.at[pl.ds(b * T + t0 + p * CT, CT)],
                ssem[s + b])
    for p in range(max(0, NTC - 2), NTC):
        for b in range(B):
            stores[(p, b)].wait()


def kernel(idx, token_embedding_table, position_embedding_table):
    idx_flat = idx.astype(jnp.int32).reshape(B * T)
    out = _embed(idx_flat, token_embedding_table, position_embedding_table)
    return out.reshape(B, T, D)
